# baseline (speedup 1.0000x reference)
import functools

import jax
import jax.numpy as jnp
from jax import lax
from jax.experimental import pallas as pl
from jax.experimental.pallas import tpu as pltpu
from jax.experimental.pallas import tpu_sc as plsc

HIDDEN = 4096
NUM_EXPERTS = 64
TOP_K = 8
BLK = 1024
TOKENS = 16384

NC = 2
NS = 16
NW = NC * NS
CHUNK = TOKENS // NW
GRP = 16


def _softmax_kernel(x_ref, w_ref, scores_ref):
    x = x_ref[...]
    w = w_ref[...]
    logits = jax.lax.dot_general(
        w, x, (((1,), (1,)), ((), ())), preferred_element_type=jnp.float32
    )
    m = jnp.max(logits, axis=0, keepdims=True)
    e = jnp.exp(logits - m)
    scores_t = e / jnp.sum(e, axis=0, keepdims=True)
    scores_ref[...] = scores_t.T


def _tc_scores(x, W):
    tokens = x.shape[0]
    return pl.pallas_call(
        _softmax_kernel,
        grid=(tokens // BLK,),
        in_specs=[
            pl.BlockSpec((BLK, HIDDEN), lambda i: (i, 0)),
            pl.BlockSpec((NUM_EXPERTS, HIDDEN), lambda i: (0, 0)),
        ],
        out_specs=pl.BlockSpec((BLK, NUM_EXPERTS), lambda i: (i, 0)),
        out_shape=jax.ShapeDtypeStruct((tokens, NUM_EXPERTS), jnp.float32),
    )(x, W)


@functools.partial(
    pl.kernel,
    mesh=plsc.VectorSubcoreMesh(core_axis_name="c", subcore_axis_name="s"),
    out_type=[
        jax.ShapeDtypeStruct((TOKENS * TOP_K,), jnp.float32),
        jax.ShapeDtypeStruct((TOKENS * TOP_K,), jnp.int32),
    ],
    scratch_types=[
        pltpu.VMEM((GRP * NUM_EXPERTS,), jnp.float32),
        pltpu.VMEM((GRP * TOP_K,), jnp.float32),
        pltpu.VMEM((GRP * TOP_K,), jnp.int32),
    ],
    compiler_params=pltpu.CompilerParams(needs_layout_passes=False),
)
def _sc_topk(scores_hbm, wts_hbm, idx_hbm, blk_v, wv, iv):
    wid = lax.axis_index("s") * NC + lax.axis_index("c")
    base = wid * CHUNK
    lanes = lax.broadcasted_iota(jnp.int32, (16,), 0)

    def group(g, _):
        off = base + g * GRP
        pltpu.sync_copy(
            scores_hbm.at[pl.ds(off * NUM_EXPERTS, GRP * NUM_EXPERTS)], blk_v
        )
        bv = [jnp.full((16,), -1.0, jnp.float32) for _ in range(TOP_K)]
        bi = [jnp.zeros((16,), jnp.int32) for _ in range(TOP_K)]
        for e in range(NUM_EXPERTS):
            c = plsc.load_gather(blk_v, [lanes * NUM_EXPERTS + e])
            ci = jnp.full((16,), e, jnp.int32)
            for j in range(TOP_K):
                take = c > bv[j]
                bv[j], c = jnp.where(take, c, bv[j]), jnp.where(take, bv[j], c)
                bi[j], ci = jnp.where(take, ci, bi[j]), jnp.where(take, bi[j], ci)
        for j in range(TOP_K):
            pos = lanes * TOP_K + j
            plsc.store_scatter(wv, [pos], bv[j])
            plsc.store_scatter(iv, [pos], bi[j])
        pltpu.sync_copy(wv, wts_hbm.at[pl.ds(off * TOP_K, GRP * TOP_K)])
        pltpu.sync_copy(iv, idx_hbm.at[pl.ds(off * TOP_K, GRP * TOP_K)])
        return 0

    lax.fori_loop(0, CHUNK // GRP, group, 0)


@jax.jit
def kernel(x, W):
    tokens = x.shape[0]
    scores = _tc_scores(x, W)
    wts_flat, idx_flat = _sc_topk(scores.reshape(-1))
    return (
        scores,
        wts_flat.reshape(tokens, TOP_K),
        idx_flat.reshape(tokens, TOP_K),
    )


# speedup vs baseline: 1.7390x; 1.7390x over previous
import functools

import jax
import jax.numpy as jnp
from jax.experimental import pallas as pl

HIDDEN = 4096
NUM_EXPERTS = 64
TOP_K = 8
BLK = 1024


def _router_kernel(x_ref, w_ref, scores_ref, wts_ref, idx_ref):
    x = x_ref[...]
    w = w_ref[...]
    logits = jax.lax.dot_general(
        w, x, (((1,), (1,)), ((), ())), preferred_element_type=jnp.float32
    )
    m = jnp.max(logits, axis=0, keepdims=True)
    e = jnp.exp(logits - m)
    scores_t = e / jnp.sum(e, axis=0, keepdims=True)
    scores_ref[...] = scores_t.T

    iota = jax.lax.broadcasted_iota(jnp.int32, scores_t.shape, 0)
    work = scores_t
    wts = []
    idxs = []
    for _ in range(TOP_K):
        mj = jnp.max(work, axis=0, keepdims=True)
        ij = jnp.min(jnp.where(work == mj, iota, NUM_EXPERTS), axis=0, keepdims=True)
        wts.append(mj)
        idxs.append(ij)
        work = jnp.where(iota == ij, -1.0, work)
    wts_ref[...] = jnp.concatenate(wts, axis=0).T
    idx_ref[...] = jnp.concatenate(idxs, axis=0).T


@jax.jit
def kernel(x, W):
    tokens = x.shape[0]
    grid = (tokens // BLK,)
    return pl.pallas_call(
        _router_kernel,
        grid=grid,
        in_specs=[
            pl.BlockSpec((BLK, HIDDEN), lambda i: (i, 0)),
            pl.BlockSpec((NUM_EXPERTS, HIDDEN), lambda i: (0, 0)),
        ],
        out_specs=[
            pl.BlockSpec((BLK, NUM_EXPERTS), lambda i: (i, 0)),
            pl.BlockSpec((BLK, TOP_K), lambda i: (i, 0)),
            pl.BlockSpec((BLK, TOP_K), lambda i: (i, 0)),
        ],
        out_shape=[
            jax.ShapeDtypeStruct((tokens, NUM_EXPERTS), jnp.float32),
            jax.ShapeDtypeStruct((tokens, TOP_K), jnp.float32),
            jax.ShapeDtypeStruct((tokens, TOP_K), jnp.int32),
        ],
    )(x, W)


# speedup vs baseline: 1.7406x; 1.0009x over previous
import jax
import jax.numpy as jnp
from jax.experimental import pallas as pl

HIDDEN = 4096
NUM_EXPERTS = 64
TOP_K = 8
BLK = 1024


def _router_kernel(x1_ref, x2_ref, w_ref, scores_ref, wts_ref, idx_ref):
    w = w_ref[...]
    logits = jax.lax.dot_general(
        w[:, : HIDDEN // 2],
        x1_ref[...],
        (((1,), (1,)), ((), ())),
        preferred_element_type=jnp.float32,
    ) + jax.lax.dot_general(
        w[:, HIDDEN // 2 :],
        x2_ref[...],
        (((1,), (1,)), ((), ())),
        preferred_element_type=jnp.float32,
    )
    m = jnp.max(logits, axis=0, keepdims=True)
    e = jnp.exp(logits - m)
    scores_t = e / jnp.sum(e, axis=0, keepdims=True)
    scores_ref[...] = scores_t.T

    iota = jax.lax.broadcasted_iota(jnp.int32, scores_t.shape, 0)
    work = scores_t
    wts = []
    idxs = []
    for _ in range(TOP_K):
        mj = jnp.max(work, axis=0, keepdims=True)
        ij = jnp.min(jnp.where(work == mj, iota, NUM_EXPERTS), axis=0, keepdims=True)
        wts.append(mj)
        idxs.append(ij)
        work = jnp.where(iota == ij, -1.0, work)
    wts_ref[...] = jnp.concatenate(wts, axis=0).T
    idx_ref[...] = jnp.concatenate(idxs, axis=0).T


@jax.jit
def kernel(x, W):
    tokens = x.shape[0]
    grid = (tokens // BLK,)
    return pl.pallas_call(
        _router_kernel,
        grid=grid,
        in_specs=[
            pl.BlockSpec((BLK, HIDDEN // 2), lambda i: (i, 0)),
            pl.BlockSpec((BLK, HIDDEN // 2), lambda i: (i, 1)),
            pl.BlockSpec((NUM_EXPERTS, HIDDEN), lambda i: (0, 0)),
        ],
        out_specs=[
            pl.BlockSpec((BLK, NUM_EXPERTS), lambda i: (i, 0)),
            pl.BlockSpec((BLK, TOP_K), lambda i: (i, 0)),
            pl.BlockSpec((BLK, TOP_K), lambda i: (i, 0)),
        ],
        out_shape=[
            jax.ShapeDtypeStruct((tokens, NUM_EXPERTS), jnp.float32),
            jax.ShapeDtypeStruct((tokens, TOP_K), jnp.float32),
            jax.ShapeDtypeStruct((tokens, TOP_K), jnp.int32),
        ],
    )(x, x, W)
